# Initial kernel scaffold; baseline (speedup 1.0000x reference)
import jax
import jax.numpy as jnp
from jax.experimental import pallas as pl


def kernel(text, category, text_table, cat_table, W1, b1, W2, b2, W3, b3):
    raise NotImplementedError("write your pallas kernel here")


# speedup vs baseline: 2.0812x; 2.0812x over previous
import functools

import jax
import jax.numpy as jnp
from jax import lax
from jax.experimental import pallas as pl
from jax.experimental.pallas import tpu as pltpu
from jax.experimental.pallas import tpu_sc as plsc

B = 16384
BAG = 50
TD = 64
CD = 32

NC = 2
NS = 16
NW = NC * NS
EPW = B // NW

CHUNK = 16
ROWS = CHUNK * BAG
GB = 100
NG = ROWS // GB
NCHUNK = EPW // CHUNK
CAT_GB = 64


def _sc_body(text_hbm, cat_hbm, ttab_hbm, ctab_hbm, sum_hbm, cemb_hbm,
             idx_v, rows_v, sum_v, cidx_v, crows_v, sem):
    c = lax.axis_index("c")
    s = lax.axis_index("s")
    wid = c * NS + s
    base = pl.multiple_of(wid * EPW, EPW)

    crow0 = pl.multiple_of(wid * (EPW // CAT_GB), EPW // CAT_GB)
    pltpu.sync_copy(cat_hbm.at[pl.ds(crow0, EPW // CAT_GB)], cidx_v)
    cds = []
    for j in range(EPW // CAT_GB):
        cds.append(pltpu.async_copy(
            ctab_hbm.at[cidx_v.at[j]],
            crows_v.at[pl.ds(j * CAT_GB, CAT_GB)], sem))
    for d in cds:
        d.wait()
    pltpu.sync_copy(crows_v, cemb_hbm.at[pl.ds(base, EPW)])

    def chunk_body(k, carry):
        ebase = pl.multiple_of(base + k * CHUNK, CHUNK)
        trow0 = pl.multiple_of((base + k * CHUNK) * BAG // GB, NG)
        pltpu.sync_copy(text_hbm.at[pl.ds(trow0, NG)], idx_v)
        ds_list = []
        for j in range(NG):
            ds_list.append(pltpu.async_copy(
                ttab_hbm.at[idx_v.at[j]],
                rows_v.at[pl.ds(j * GB, GB)], sem))
        for d in ds_list:
            d.wait()

        def elem_body(e, carry2):
            r0 = e * BAG
            for d in range(TD // 16):
                acc = rows_v[r0, pl.ds(d * 16, 16)]
                for l in range(1, BAG):
                    acc = acc + rows_v[r0 + l, pl.ds(d * 16, 16)]
                sum_v[e, pl.ds(d * 16, 16)] = acc
            return carry2

        lax.fori_loop(0, CHUNK, elem_body, 0)
        pltpu.sync_copy(sum_v, sum_hbm.at[pl.ds(ebase, CHUNK)])
        return carry

    lax.fori_loop(0, NCHUNK, chunk_body, 0)


_sc_gather = functools.partial(
    pl.kernel,
    out_type=[
        jax.ShapeDtypeStruct((B, TD), jnp.float32),
        jax.ShapeDtypeStruct((B, CD), jnp.float32),
    ],
    mesh=plsc.VectorSubcoreMesh(core_axis_name="c", subcore_axis_name="s"),
    compiler_params=pltpu.CompilerParams(use_tc_tiling_on_sc=False),
    scratch_types=[
        pltpu.VMEM((NG, GB), jnp.int32),
        pltpu.VMEM((ROWS, TD), jnp.float32),
        pltpu.VMEM((CHUNK, TD), jnp.float32),
        pltpu.VMEM((EPW // CAT_GB, CAT_GB), jnp.int32),
        pltpu.VMEM((EPW, CD), jnp.float32),
        pltpu.SemaphoreType.DMA,
    ],
)(_sc_body)


MLP_BLK = 2048


def _mlp_body(s_ref, c_ref, t_ref, a1_ref, a2_ref, w2_ref, w3_ref,
              b1_ref, b2_ref, b3_ref, o_ref):
    cnt = jnp.sum((t_ref[...] != 0).astype(jnp.float32), axis=1,
                  keepdims=True)
    t = s_ref[...] / jnp.maximum(cnt, 1.0)
    hp = jax.lax.Precision.HIGHEST
    h = jnp.dot(t, a1_ref[...], precision=hp)
    h = h + jnp.dot(c_ref[...], a2_ref[...], precision=hp)
    h = jnp.maximum(h + b1_ref[...], 0.0)
    h = jnp.maximum(jnp.dot(h, w2_ref[...], precision=hp) + b2_ref[...], 0.0)
    o_ref[...] = jnp.dot(h, w3_ref[...], precision=hp) + b3_ref[...]


def _mlp(sums, cemb, text, a1, a2, w2t, w3t, b1, b2, b3):
    grid = B // MLP_BLK
    h1 = b1.shape[-1]
    h2 = b2.shape[-1]
    return pl.pallas_call(
        _mlp_body,
        grid=(grid,),
        in_specs=[
            pl.BlockSpec((MLP_BLK, TD), lambda i: (i, 0)),
            pl.BlockSpec((MLP_BLK, CD), lambda i: (i, 0)),
            pl.BlockSpec((MLP_BLK, BAG), lambda i: (i, 0)),
            pl.BlockSpec((TD, h1), lambda i: (0, 0)),
            pl.BlockSpec((CD, h1), lambda i: (0, 0)),
            pl.BlockSpec((h1, h2), lambda i: (0, 0)),
            pl.BlockSpec((h2, TD), lambda i: (0, 0)),
            pl.BlockSpec((1, h1), lambda i: (0, 0)),
            pl.BlockSpec((1, h2), lambda i: (0, 0)),
            pl.BlockSpec((1, TD), lambda i: (0, 0)),
        ],
        out_specs=pl.BlockSpec((MLP_BLK, TD), lambda i: (i, 0)),
        out_shape=jax.ShapeDtypeStruct((B, TD), jnp.float32),
    )(sums, cemb, text, a1, a2, w2t, w3t, b1, b2, b3)


def kernel(text, category, text_table, cat_table, W1, b1, W2, b2, W3, b3):
    text = text.astype(jnp.int32)
    category = category.astype(jnp.int32)
    text2d = text.reshape(B * BAG // GB, GB)
    cat2d = category.reshape(B // CAT_GB, CAT_GB)
    sums, cemb = _sc_gather(text2d, cat2d, text_table, cat_table)
    a1 = W1.T[:TD, :]
    a2 = W1.T[TD:, :]
    return _mlp(sums, cemb, text, a1, a2, W2.T, W3.T,
                b1.reshape(1, -1), b2.reshape(1, -1), b3.reshape(1, -1))
